# Initial kernel scaffold; baseline (speedup 1.0000x reference)
import jax
import jax.numpy as jnp
from jax.experimental import pallas as pl


def kernel(x, embedding):
    raise NotImplementedError("write your pallas kernel here")


# speedup vs baseline: 1.5024x; 1.5024x over previous
import jax
import jax.numpy as jnp
from jax.experimental import pallas as pl

S_BLK = 256


def _body(x_ref, emb_ref, out_ref):
    out_ref[0] = x_ref[0] + emb_ref[...]


def kernel(x, embedding):
    batch, seq_len, d_model = x.shape
    n_seq = seq_len // S_BLK
    return pl.pallas_call(
        _body,
        grid=(n_seq, batch),
        in_specs=[
            pl.BlockSpec((1, S_BLK, d_model), lambda i, b: (b, i, 0)),
            pl.BlockSpec((S_BLK, d_model), lambda i, b: (i, 0)),
        ],
        out_specs=pl.BlockSpec((1, S_BLK, d_model), lambda i, b: (b, i, 0)),
        out_shape=jax.ShapeDtypeStruct(x.shape, x.dtype),
    )(x, embedding)
